# Initial kernel scaffold; baseline (speedup 1.0000x reference)
import functools

import jax
import jax.numpy as jnp
from jax import lax
from jax.experimental import pallas as pl
from jax.experimental.pallas import tpu as pltpu
from jax.experimental.pallas import tpu_sc as plsc

N_TAG = 1000
N_VID = 10000
D = 512
E = 75000

NC, NS, L = 2, 16, 16
EPAD = 75008
EPT = EPAD // NS
VREGS = EPT // L
IROWS = 37
IDX_ROWS = 2 * IROWS
R = 2000
CHUNKS = N_VID // R
SLAB = R * N_TAG
PAD_PER_TILE = 4096
SLAB_TOTAL = SLAB + NS * PAD_PER_TILE
WPT = SLAB // NS
ZBUF = 25000


def _sc_body(ps, pd, ng, nd, labels_out, cntall_out,
             slab, psv, pdv, nsv, ndv, fpos, fneg, idx2, ones_v, zeros_v,
             sem_a, sem_scat):
    cid = lax.axis_index("c")
    sid = lax.axis_index("s")
    ebase = sid * EPT
    iota = lax.iota(jnp.int32, L)
    dumbase = SLAB + sid * PAD_PER_TILE

    hs = [pltpu.async_copy(ps.at[pl.ds(ebase, EPT)], psv, sem_a),
          pltpu.async_copy(pd.at[pl.ds(ebase, EPT)], pdv, sem_a),
          pltpu.async_copy(ng.at[pl.ds(ebase, EPT)], nsv, sem_a),
          pltpu.async_copy(nd.at[pl.ds(ebase, EPT)], ndv, sem_a)]
    for h in hs:
        h.wait()

    ones16 = jnp.ones((L,), jnp.float32)
    zero16 = jnp.zeros((L,), jnp.float32)
    for t in range(8):
        ones_v[pl.ds(t * L, L)] = ones16

    def zfill(j, c):
        zeros_v[pl.ds(j * L, L)] = zero16
        return c
    lax.fori_loop(0, ZBUF // L, zfill, 0)
    zeros_v[pl.ds(ZBUF - L, L)] = zero16

    def flatten(j, c):
        fpos[pl.ds(j * L, L)] = pdv[pl.ds(j * L, L)] * N_TAG + psv[pl.ds(j * L, L)]
        fneg[pl.ds(j * L, L)] = ndv[pl.ds(j * L, L)] * N_TAG + nsv[pl.ds(j * L, L)]
        return c
    lax.fori_loop(0, VREGS, flatten, 0)

    def build_idx(buf, row0, lo):
        hi = lo + SLAB

        def bd(j, c):
            f = buf[pl.ds(j * L, L)]
            ok = (f >= lo) & (f < hi)
            dum = dumbase + ((j * L) & (PAD_PER_TILE - 1)) + iota
            idx2[row0 + (j >> 3), pl.ds((j & 7) * L, L)] = jnp.where(ok, f - lo, dum)
            return c
        lax.fori_loop(0, VREGS, bd, 0)
        for t in range(VREGS, IROWS * 8):
            idx2[row0 + t // 8, pl.ds((t % 8) * L, L)] = (
                dumbase + ((t * L) & (PAD_PER_TILE - 1)) + iota)

    def run_task(target, chunk, include_neg):
        zh = [pltpu.async_copy(zeros_v, slab.at[pl.ds(sid * WPT + k * ZBUF, ZBUF)], sem_a)
              for k in range(WPT // ZBUF)]
        for h in zh:
            h.wait()
        plsc.subcore_barrier()

        build_idx(fpos, 0, chunk * SLAB)
        if include_neg:
            build_idx(fneg, IROWS, chunk * SLAB)
        nrows = IDX_ROWS if include_neg else IROWS
        sh = [pltpu.async_copy(ones_v, slab.at[idx2.at[r]], sem_scat, add=True)
              for r in range(nrows)]
        for h in sh:
            h.wait()
        plsc.subcore_barrier()

        pltpu.async_copy(slab.at[pl.ds(sid * WPT, WPT)],
                         target.at[pl.ds(chunk * SLAB + sid * WPT, WPT)],
                         sem_a).wait()
        plsc.subcore_barrier()

    @pl.when(cid == 0)
    def _():
        for ch in (0, 2, 4):
            run_task(cntall_out, ch, True)
        for ch in (1, 3):
            run_task(labels_out, ch, False)

    @pl.when(cid == 1)
    def _():
        for ch in (1, 3):
            run_task(cntall_out, ch, True)
        for ch in (0, 2, 4):
            run_task(labels_out, ch, False)


_sc_counts = pl.kernel(
    _sc_body,
    out_type=[jax.ShapeDtypeStruct((N_VID * N_TAG,), jnp.float32),
              jax.ShapeDtypeStruct((N_VID * N_TAG,), jnp.float32)],
    mesh=plsc.VectorSubcoreMesh(core_axis_name="c", subcore_axis_name="s",
                                num_cores=NC, num_subcores=NS),
    scratch_types=[
        pltpu.VMEM_SHARED((SLAB_TOTAL,), jnp.float32),
        pltpu.VMEM((EPT,), jnp.int32),
        pltpu.VMEM((EPT,), jnp.int32),
        pltpu.VMEM((EPT,), jnp.int32),
        pltpu.VMEM((EPT,), jnp.int32),
        pltpu.VMEM((EPT,), jnp.int32),
        pltpu.VMEM((EPT,), jnp.int32),
        pltpu.VMEM((IDX_ROWS + 6, 128), jnp.int32),
        pltpu.VMEM((128,), jnp.float32),
        pltpu.VMEM((ZBUF,), jnp.float32),
        pltpu.SemaphoreType.DMA,
        pltpu.SemaphoreType.DMA,
    ],
)


BM = 400


def _tc_body(hv_ref, ht_ref, cnt_ref, out_ref):
    acc = lax.dot_general(hv_ref[...], ht_ref[...], (((1,), (1,)), ((), ())),
                          preferred_element_type=jnp.float32,
                          precision=lax.Precision.HIGHEST)
    out_ref[...] = acc * cnt_ref[...]


def _tc_score(h_video, h_tag, cnt):
    return pl.pallas_call(
        _tc_body,
        grid=(N_VID // BM,),
        in_specs=[pl.BlockSpec((BM, D), lambda i: (i, 0)),
                  pl.BlockSpec((N_TAG, D), lambda i: (0, 0)),
                  pl.BlockSpec((BM, N_TAG), lambda i: (i, 0))],
        out_specs=pl.BlockSpec((BM, N_TAG), lambda i: (i, 0)),
        out_shape=jax.ShapeDtypeStruct((N_VID, N_TAG), jnp.float32),
    )(h_video, h_tag, cnt)


def kernel(h_tag, h_video, pos_src, pos_dst, neg_src, neg_dst):
    npad = EPAD - E
    pz = jnp.zeros((npad,), jnp.int32)
    pv = jnp.full((npad,), N_VID, jnp.int32)
    ps = jnp.concatenate([pos_src.astype(jnp.int32), pz])
    pd = jnp.concatenate([pos_dst.astype(jnp.int32), pv])
    ng = jnp.concatenate([neg_src.astype(jnp.int32), pz])
    nd = jnp.concatenate([neg_dst.astype(jnp.int32), pv])

    labels_flat, cntall_flat = _sc_counts(ps, pd, ng, nd)
    labels = labels_flat.reshape(N_VID, N_TAG)
    cnt = cntall_flat.reshape(N_VID, N_TAG)
    cls = _tc_score(h_video, h_tag, cnt)
    return cls, labels


# speedup vs baseline: 5.1672x; 5.1672x over previous
import jax
import jax.numpy as jnp
from jax import lax
from jax.experimental import pallas as pl
from jax.experimental.pallas import tpu as pltpu
from jax.experimental.pallas import tpu_sc as plsc

N_TAG = 1000
N_VID = 10000
D = 512
E = 75000

NC, NS, L = 2, 16, 16
EPAD = 75008
EPT = EPAD // NS
VREGS = EPT // L
IROWS = 37
IDX_ROWS = 2 * IROWS
R = 960
CHUNKS = 11
SLAB = R * N_TAG
PAD_PER_TILE = 1024
SLAB_TOTAL = SLAB + NS * PAD_PER_TILE
WPT = SLAB // NS
ZB = 4096

SEGS = []
_off = 0
while _off < WPT:
    _n = min(ZB, WPT - _off)
    SEGS.append((_off, _n))
    _off += _n
assert all(o % 8 == 0 and n % 8 == 0 for o, n in SEGS)


def _sc_body(ps, pd, ng, nd, labels_out, cntall_out,
             slab, fpos, fneg, temp, idx2, ones_v, zeros_v,
             bounce_a, bounce_b, sem_a, sem_scat, sem_out):
    bounce = (bounce_a, bounce_b)
    cid = lax.axis_index("c")
    sid = lax.axis_index("s")
    ebase = sid * EPT
    iota = lax.iota(jnp.int32, L)
    dumbase = SLAB + sid * PAD_PER_TILE

    ones16 = jnp.ones((L,), jnp.float32)
    zero16 = jnp.zeros((L,), jnp.float32)
    for t in range(128 // L):
        ones_v[pl.ds(t * L, L)] = ones16

    def zfill(j, c):
        zeros_v[pl.ds(j * L, L)] = zero16
        return c
    lax.fori_loop(0, ZB // L, zfill, 0)

    def load_flat(src_hbm, dst_hbm, out_ref):
        h1 = pltpu.async_copy(src_hbm.at[pl.ds(ebase, EPT)], out_ref, sem_a)
        h2 = pltpu.async_copy(dst_hbm.at[pl.ds(ebase, EPT)], temp, sem_a)
        h1.wait()
        h2.wait()

        def flat(j, c):
            out_ref[pl.ds(j * L, L)] = (temp[pl.ds(j * L, L)] * N_TAG
                                        + out_ref[pl.ds(j * L, L)])
            return c
        lax.fori_loop(0, VREGS, flat, 0)

    load_flat(ps, pd, fpos)
    load_flat(ng, nd, fneg)

    def build_idx(buf, row0, lo):
        hi = lo + SLAB

        def bd(j, c):
            f = buf[pl.ds(j * L, L)]
            ok = (f >= lo) & (f < hi)
            dum = dumbase + ((j * L) & (PAD_PER_TILE - 1)) + iota
            idx2[row0 + (j >> 3), pl.ds((j & 7) * L, L)] = jnp.where(ok, f - lo, dum)
            return c
        lax.fori_loop(0, VREGS, bd, 0)
        for t in range(VREGS, IROWS * 8):
            idx2[row0 + t // 8, pl.ds((t % 8) * L, L)] = (
                dumbase + ((t * L) & (PAD_PER_TILE - 1)) + iota)

    def run_phase(target, include_neg, parity):
        def task(t, c):
            chunk = 2 * t + parity
            lo = jnp.minimum(chunk * SLAB, N_VID * N_TAG - SLAB)

            zh = [pltpu.async_copy(zeros_v.at[pl.ds(0, n)],
                                   slab.at[pl.ds(sid * WPT + o, n)], sem_a)
                  for o, n in SEGS]
            for h in zh:
                h.wait()
            plsc.subcore_barrier()

            build_idx(fpos, 0, lo)
            if include_neg:
                build_idx(fneg, IROWS, lo)
            nrows = IDX_ROWS if include_neg else IROWS

            def fire(r, c2):
                pltpu.async_copy(ones_v, slab.at[idx2.at[r]], sem_scat, add=True)
                return c2
            lax.fori_loop(0, nrows, fire, 0)

            def drain(r, c2):
                pltpu.make_async_copy(ones_v, slab.at[idx2.at[0]], sem_scat).wait()
                return c2
            lax.fori_loop(0, nrows, drain, 0)
            plsc.subcore_barrier()

            houts = [None, None]
            for k, (o, n) in enumerate(SEGS):
                b = k % 2
                if houts[b] is not None:
                    houts[b].wait()
                pltpu.async_copy(slab.at[pl.ds(sid * WPT + o, n)],
                                 bounce[b].at[pl.ds(0, n)], sem_a).wait()
                houts[b] = pltpu.async_copy(
                    bounce[b].at[pl.ds(0, n)],
                    target.at[pl.ds(lo + sid * WPT + o, n)],
                    sem_out)
            for h in houts:
                h.wait()
            plsc.subcore_barrier()
            return c

        lax.fori_loop(0, (CHUNKS + 1 - parity) // 2, task, 0)

    run_phase(cntall_out, True, cid)
    run_phase(labels_out, False, 1 - cid)


_sc_counts = pl.kernel(
    _sc_body,
    out_type=[jax.ShapeDtypeStruct((N_VID * N_TAG,), jnp.float32),
              jax.ShapeDtypeStruct((N_VID * N_TAG,), jnp.float32)],
    mesh=plsc.VectorSubcoreMesh(core_axis_name="c", subcore_axis_name="s",
                                num_cores=NC, num_subcores=NS),
    scratch_types=[
        pltpu.VMEM_SHARED((SLAB_TOTAL,), jnp.float32),
        pltpu.VMEM((EPT,), jnp.int32),
        pltpu.VMEM((EPT,), jnp.int32),
        pltpu.VMEM((EPT,), jnp.int32),
        pltpu.VMEM((IDX_ROWS + 6, 128), jnp.int32),
        pltpu.VMEM((128,), jnp.float32),
        pltpu.VMEM((ZB,), jnp.float32),
        pltpu.VMEM((ZB,), jnp.float32),
        pltpu.VMEM((ZB,), jnp.float32),
        pltpu.SemaphoreType.DMA,
        pltpu.SemaphoreType.DMA,
        pltpu.SemaphoreType.DMA,
    ],
)


BM = 400


def _tc_body(hv_ref, ht_ref, cnt_ref, out_ref):
    acc = lax.dot_general(hv_ref[...], ht_ref[...], (((1,), (1,)), ((), ())),
                          preferred_element_type=jnp.float32,
                          precision=lax.Precision.HIGHEST)
    out_ref[...] = acc * cnt_ref[...]


def _tc_score(h_video, h_tag, cnt):
    return pl.pallas_call(
        _tc_body,
        grid=(N_VID // BM,),
        in_specs=[pl.BlockSpec((BM, D), lambda i: (i, 0)),
                  pl.BlockSpec((N_TAG, D), lambda i: (0, 0)),
                  pl.BlockSpec((BM, N_TAG), lambda i: (i, 0))],
        out_specs=pl.BlockSpec((BM, N_TAG), lambda i: (i, 0)),
        out_shape=jax.ShapeDtypeStruct((N_VID, N_TAG), jnp.float32),
    )(h_video, h_tag, cnt)


def kernel(h_tag, h_video, pos_src, pos_dst, neg_src, neg_dst):
    npad = EPAD - E
    pz = jnp.zeros((npad,), jnp.int32)
    pv = jnp.full((npad,), N_VID, jnp.int32)
    ps = jnp.concatenate([pos_src.astype(jnp.int32), pz])
    pd = jnp.concatenate([pos_dst.astype(jnp.int32), pv])
    ng = jnp.concatenate([neg_src.astype(jnp.int32), pz])
    nd = jnp.concatenate([neg_dst.astype(jnp.int32), pv])

    labels_flat, cntall_flat = _sc_counts(ps, pd, ng, nd)
    labels = labels_flat.reshape(N_VID, N_TAG)
    cnt = cntall_flat.reshape(N_VID, N_TAG)
    cls = _tc_score(h_video, h_tag, cnt)
    return cls, labels


# speedup vs baseline: 5.8582x; 1.1337x over previous
import jax
import jax.numpy as jnp
from jax import lax
from jax.experimental import pallas as pl
from jax.experimental.pallas import tpu as pltpu
from jax.experimental.pallas import tpu_sc as plsc

N_TAG = 1000
N_VID = 10000
D = 512
E = 75000

NC, NS, L = 2, 16, 16
EPAD = 75008
EPT = EPAD // NS
VREGS = EPT // L
IROWS = 37
IDX_ROWS = 2 * IROWS
NT_PAD = 1024
R = 960
CHUNKS = 11
SLAB = R * NT_PAD
PAD_PER_TILE = 1024
SLAB_TOTAL = SLAB + NS * PAD_PER_TILE
FLAT = N_VID * NT_PAD
WPT = SLAB // NS
ZB = 4096

SEGS = []
_off = 0
while _off < WPT:
    _n = min(ZB, WPT - _off)
    SEGS.append((_off, _n))
    _off += _n
assert all(o % 8 == 0 and n % 8 == 0 for o, n in SEGS)


def _sc_body(ps, pd, ng, nd, labels_out, cntall_out,
             slab, fpos, fneg, temp, idx2, ones_v, zeros_v,
             bounce_a, bounce_b, sem_a, sem_scat, sem_out):
    bounce = (bounce_a, bounce_b)
    cid = lax.axis_index("c")
    sid = lax.axis_index("s")
    ebase = sid * EPT
    iota = lax.iota(jnp.int32, L)
    dumbase = SLAB + sid * PAD_PER_TILE

    ones16 = jnp.ones((L,), jnp.float32)
    zero16 = jnp.zeros((L,), jnp.float32)
    for t in range(128 // L):
        ones_v[pl.ds(t * L, L)] = ones16

    def zfill(j, c):
        zeros_v[pl.ds(j * L, L)] = zero16
        return c
    lax.fori_loop(0, ZB // L, zfill, 0)

    def load_flat(src_hbm, dst_hbm, out_ref):
        h1 = pltpu.async_copy(src_hbm.at[pl.ds(ebase, EPT)], out_ref, sem_a)
        h2 = pltpu.async_copy(dst_hbm.at[pl.ds(ebase, EPT)], temp, sem_a)
        h1.wait()
        h2.wait()

        def flat(j, c):
            t = out_ref[pl.ds(j * L, L)]
            v = temp[pl.ds(j * L, L)]
            out_ref[pl.ds(j * L, L)] = (v << 10) + t
            return c
        lax.fori_loop(0, VREGS, flat, 0)

    load_flat(ps, pd, fpos)
    load_flat(ng, nd, fneg)

    def build_idx(buf, row0, lo):
        hi = lo + SLAB

        def bd(j, c):
            f = buf[pl.ds(j * L, L)]
            ok = (f >= lo) & (f < hi)
            dum = dumbase + ((j * L) & (PAD_PER_TILE - 1)) + iota
            idx2[row0 + (j >> 3), pl.ds((j & 7) * L, L)] = jnp.where(ok, f - lo, dum)
            return c
        lax.fori_loop(0, VREGS, bd, 0)
        for t in range(VREGS, IROWS * 8):
            idx2[row0 + t // 8, pl.ds((t % 8) * L, L)] = (
                dumbase + ((t * L) & (PAD_PER_TILE - 1)) + iota)

    def run_phase(target, include_neg, parity):
        def task(t, c):
            chunk = 2 * t + parity
            lo = jnp.minimum(chunk * SLAB, FLAT - SLAB)

            zh = [pltpu.async_copy(zeros_v.at[pl.ds(0, n)],
                                   slab.at[pl.ds(sid * WPT + o, n)], sem_a)
                  for o, n in SEGS]
            for h in zh:
                h.wait()
            plsc.subcore_barrier()

            build_idx(fpos, 0, lo)
            if include_neg:
                build_idx(fneg, IROWS, lo)
            nrows = IDX_ROWS if include_neg else IROWS

            def fire(r, c2):
                pltpu.async_copy(ones_v, slab.at[idx2.at[r]], sem_scat, add=True)
                return c2
            lax.fori_loop(0, nrows, fire, 0)

            def drain(r, c2):
                pltpu.make_async_copy(ones_v, slab.at[idx2.at[0]], sem_scat).wait()
                return c2
            lax.fori_loop(0, nrows, drain, 0)
            plsc.subcore_barrier()

            houts = [None, None]
            for k, (o, n) in enumerate(SEGS):
                b = k % 2
                if houts[b] is not None:
                    houts[b].wait()
                pltpu.async_copy(slab.at[pl.ds(sid * WPT + o, n)],
                                 bounce[b].at[pl.ds(0, n)], sem_a).wait()
                houts[b] = pltpu.async_copy(
                    bounce[b].at[pl.ds(0, n)],
                    target.at[pl.ds(lo + sid * WPT + o, n)],
                    sem_out)
            for h in houts:
                h.wait()
            plsc.subcore_barrier()
            return c

        lax.fori_loop(0, (CHUNKS + 1 - parity) // 2, task, 0)

    run_phase(cntall_out, True, cid)
    run_phase(labels_out, False, 1 - cid)


_sc_counts = pl.kernel(
    _sc_body,
    out_type=[jax.ShapeDtypeStruct((FLAT,), jnp.float32),
              jax.ShapeDtypeStruct((FLAT,), jnp.float32)],
    mesh=plsc.VectorSubcoreMesh(core_axis_name="c", subcore_axis_name="s",
                                num_cores=NC, num_subcores=NS),
    scratch_types=[
        pltpu.VMEM_SHARED((SLAB_TOTAL,), jnp.float32),
        pltpu.VMEM((EPT,), jnp.int32),
        pltpu.VMEM((EPT,), jnp.int32),
        pltpu.VMEM((EPT,), jnp.int32),
        pltpu.VMEM((IDX_ROWS + 6, 128), jnp.int32),
        pltpu.VMEM((128,), jnp.float32),
        pltpu.VMEM((ZB,), jnp.float32),
        pltpu.VMEM((ZB,), jnp.float32),
        pltpu.VMEM((ZB,), jnp.float32),
        pltpu.SemaphoreType.DMA,
        pltpu.SemaphoreType.DMA,
        pltpu.SemaphoreType.DMA,
    ],
)


BM = 256


FB = BM * NT_PAD // 128


def _tc_body(hv_ref, ht_ref, cnt_ref, lab_ref, out_ref, lab2_ref):
    acc = lax.dot_general(hv_ref[...], ht_ref[...], (((1,), (1,)), ((), ())),
                          preferred_element_type=jnp.float32,
                          precision=lax.Precision.HIGHEST)
    cm = cnt_ref[...].reshape(BM, NT_PAD)
    out_ref[...] = (acc * cm)[:, :N_TAG]
    lab2_ref[...] = lab_ref[...].reshape(BM, NT_PAD)[:, :N_TAG]


def _tc_score(h_video, h_tag_pad, cnt_flat, lab_flat):
    return pl.pallas_call(
        _tc_body,
        grid=(pl.cdiv(N_VID, BM),),
        in_specs=[pl.BlockSpec((BM, D), lambda i: (i, 0)),
                  pl.BlockSpec((NT_PAD, D), lambda i: (0, 0)),
                  pl.BlockSpec((FB, 128), lambda i: (i, 0)),
                  pl.BlockSpec((FB, 128), lambda i: (i, 0))],
        out_specs=[pl.BlockSpec((BM, N_TAG), lambda i: (i, 0)),
                   pl.BlockSpec((BM, N_TAG), lambda i: (i, 0))],
        out_shape=[jax.ShapeDtypeStruct((N_VID, N_TAG), jnp.float32),
                   jax.ShapeDtypeStruct((N_VID, N_TAG), jnp.float32)],
    )(h_video, h_tag_pad, cnt_flat.reshape(-1, 128), lab_flat.reshape(-1, 128))


def kernel(h_tag, h_video, pos_src, pos_dst, neg_src, neg_dst):
    npad = EPAD - E
    pz = jnp.zeros((npad,), jnp.int32)
    pv = jnp.full((npad,), N_VID, jnp.int32)
    ps = jnp.concatenate([pos_src.astype(jnp.int32), pz])
    pd = jnp.concatenate([pos_dst.astype(jnp.int32), pv])
    ng = jnp.concatenate([neg_src.astype(jnp.int32), pz])
    nd = jnp.concatenate([neg_dst.astype(jnp.int32), pv])

    labels_flat, cntall_flat = _sc_counts(ps, pd, ng, nd)
    h_tag_pad = jnp.pad(h_tag, ((0, NT_PAD - N_TAG), (0, 0)))
    cls, labels = _tc_score(h_video, h_tag_pad, cntall_flat, labels_flat)
    return cls, labels


# speedup vs baseline: 7.8928x; 1.3473x over previous
import jax
import jax.numpy as jnp
from jax import lax
from jax.experimental import pallas as pl
from jax.experimental.pallas import tpu as pltpu
from jax.experimental.pallas import tpu_sc as plsc

N_TAG = 1000
N_VID = 10000
D = 512
E = 75000

NC, NS, L = 2, 16, 16
EPAD = 75008
EPT = EPAD // NS
VREGS = EPT // L
IROWS = 37
IDX_ROWS = 2 * IROWS
NV_PAD = 10240
SLAB = 983040
CHUNKS = 11
PAD_PER_TILE = 1024
SLAB_TOTAL = SLAB + NS * PAD_PER_TILE
FLAT = N_TAG * NV_PAD
WPT = SLAB // NS
ZB = 4096

SEGS = []
_off = 0
while _off < WPT:
    _n = min(ZB, WPT - _off)
    SEGS.append((_off, _n))
    _off += _n
assert all(o % 8 == 0 and n % 8 == 0 for o, n in SEGS)


def _sc_body(ps, pd, ng, nd, labels_out, cntall_out,
             slab, fpos, fneg, temp, idx2, ones_v, zeros_v,
             bounce_a, bounce_b, sem_a, sem_scat, sem_out):
    bounce = (bounce_a, bounce_b)
    cid = lax.axis_index("c")
    sid = lax.axis_index("s")
    ebase = sid * EPT
    iota = lax.iota(jnp.int32, L)
    dumbase = SLAB + sid * PAD_PER_TILE

    ones16 = jnp.ones((L,), jnp.float32)
    zero16 = jnp.zeros((L,), jnp.float32)
    for t in range(128 // L):
        ones_v[pl.ds(t * L, L)] = ones16

    def zfill(j, c):
        zeros_v[pl.ds(j * L, L)] = zero16
        return c
    lax.fori_loop(0, ZB // L, zfill, 0)

    def load_flat(src_hbm, dst_hbm, out_ref):
        h1 = pltpu.async_copy(src_hbm.at[pl.ds(ebase, EPT)], out_ref, sem_a)
        h2 = pltpu.async_copy(dst_hbm.at[pl.ds(ebase, EPT)], temp, sem_a)
        h1.wait()
        h2.wait()

        def flat(j, c):
            t = out_ref[pl.ds(j * L, L)]
            v = temp[pl.ds(j * L, L)]
            out_ref[pl.ds(j * L, L)] = t * NV_PAD + v
            return c
        lax.fori_loop(0, VREGS, flat, 0)

    load_flat(ps, pd, fpos)
    load_flat(ng, nd, fneg)

    def build_idx(buf, row0, lo):
        hi = lo + SLAB

        def bd(j, c):
            f = buf[pl.ds(j * L, L)]
            ok = (f >= lo) & (f < hi)
            dum = dumbase + ((j * L) & (PAD_PER_TILE - 1)) + iota
            idx2[row0 + (j >> 3), pl.ds((j & 7) * L, L)] = jnp.where(ok, f - lo, dum)
            return c
        lax.fori_loop(0, VREGS, bd, 0)
        for t in range(VREGS, IROWS * 8):
            idx2[row0 + t // 8, pl.ds((t % 8) * L, L)] = (
                dumbase + ((t * L) & (PAD_PER_TILE - 1)) + iota)

    def run_phase(target, include_neg, parity):
        def task(t, c):
            chunk = 2 * t + parity
            lo = jnp.minimum(chunk * SLAB, FLAT - SLAB)

            zh = [pltpu.async_copy(zeros_v.at[pl.ds(0, n)],
                                   slab.at[pl.ds(sid * WPT + o, n)], sem_a)
                  for o, n in SEGS]
            for h in zh:
                h.wait()
            plsc.subcore_barrier()

            build_idx(fpos, 0, lo)
            if include_neg:
                build_idx(fneg, IROWS, lo)
            nrows = IDX_ROWS if include_neg else IROWS

            def fire(r, c2):
                pltpu.async_copy(ones_v, slab.at[idx2.at[r]], sem_scat, add=True)
                return c2
            lax.fori_loop(0, nrows, fire, 0)

            def drain(r, c2):
                pltpu.make_async_copy(ones_v, slab.at[idx2.at[0]], sem_scat).wait()
                return c2
            lax.fori_loop(0, nrows, drain, 0)
            plsc.subcore_barrier()

            houts = [None, None]
            for k, (o, n) in enumerate(SEGS):
                b = k % 2
                if houts[b] is not None:
                    houts[b].wait()
                pltpu.async_copy(slab.at[pl.ds(sid * WPT + o, n)],
                                 bounce[b].at[pl.ds(0, n)], sem_a).wait()
                houts[b] = pltpu.async_copy(
                    bounce[b].at[pl.ds(0, n)],
                    target.at[pl.ds(lo + sid * WPT + o, n)],
                    sem_out)
            for h in houts:
                h.wait()
            plsc.subcore_barrier()
            return c

        lax.fori_loop(0, (CHUNKS + 1 - parity) // 2, task, 0)

    run_phase(cntall_out, True, cid)
    run_phase(labels_out, False, 1 - cid)


_sc_counts = pl.kernel(
    _sc_body,
    out_type=[jax.ShapeDtypeStruct((FLAT,), jnp.float32),
              jax.ShapeDtypeStruct((FLAT,), jnp.float32)],
    mesh=plsc.VectorSubcoreMesh(core_axis_name="c", subcore_axis_name="s",
                                num_cores=NC, num_subcores=NS),
    scratch_types=[
        pltpu.VMEM_SHARED((SLAB_TOTAL,), jnp.float32),
        pltpu.VMEM((EPT,), jnp.int32),
        pltpu.VMEM((EPT,), jnp.int32),
        pltpu.VMEM((EPT,), jnp.int32),
        pltpu.VMEM((IDX_ROWS + 6, 128), jnp.int32),
        pltpu.VMEM((128,), jnp.float32),
        pltpu.VMEM((ZB,), jnp.float32),
        pltpu.VMEM((ZB,), jnp.float32),
        pltpu.VMEM((ZB,), jnp.float32),
        pltpu.SemaphoreType.DMA,
        pltpu.SemaphoreType.DMA,
        pltpu.SemaphoreType.DMA,
    ],
)


BM = 1024
BV = BM // 128


def _tc_body(hv_ref, ht_ref, cnt_ref, lab_ref, out_ref, lab2_ref):
    acc = lax.dot_general(ht_ref[...], hv_ref[...], (((1,), (1,)), ((), ())),
                          preferred_element_type=jnp.float32,
                          precision=lax.Precision.HIGHEST)
    out_ref[...] = acc * cnt_ref[...].reshape(N_TAG, BM)
    lab2_ref[...] = lab_ref[...].reshape(N_TAG, BM)


def _tc_score(h_video, h_tag, cnt_flat, lab_flat):
    cnt3 = cnt_flat.reshape(N_TAG, NV_PAD // 128, 128)
    lab3 = lab_flat.reshape(N_TAG, NV_PAD // 128, 128)
    return pl.pallas_call(
        _tc_body,
        grid=(NV_PAD // BM,),
        in_specs=[pl.BlockSpec((BM, D), lambda i: (i, 0)),
                  pl.BlockSpec((N_TAG, D), lambda i: (0, 0)),
                  pl.BlockSpec((N_TAG, BV, 128), lambda i: (0, i, 0)),
                  pl.BlockSpec((N_TAG, BV, 128), lambda i: (0, i, 0))],
        out_specs=[pl.BlockSpec((N_TAG, BM), lambda i: (0, i)),
                   pl.BlockSpec((N_TAG, BM), lambda i: (0, i))],
        out_shape=[jax.ShapeDtypeStruct((N_TAG, N_VID), jnp.float32),
                   jax.ShapeDtypeStruct((N_TAG, N_VID), jnp.float32)],
    )(h_video, h_tag, cnt3, lab3)


def kernel(h_tag, h_video, pos_src, pos_dst, neg_src, neg_dst):
    npad = EPAD - E
    pz = jnp.zeros((npad,), jnp.int32)
    pv = jnp.full((npad,), N_VID, jnp.int32)
    ps = jnp.concatenate([pos_src.astype(jnp.int32), pz])
    pd = jnp.concatenate([pos_dst.astype(jnp.int32), pv])
    ng = jnp.concatenate([neg_src.astype(jnp.int32), pz])
    nd = jnp.concatenate([neg_dst.astype(jnp.int32), pv])

    labels_flat, cntall_flat = _sc_counts(ps, pd, ng, nd)
    cls_t, labels_t = _tc_score(h_video, h_tag, cntall_flat, labels_flat)
    return cls_t.T, labels_t.T


# speedup vs baseline: 9.0088x; 1.1414x over previous
import jax
import jax.numpy as jnp
from jax import lax
from jax.experimental import pallas as pl
from jax.experimental.pallas import tpu as pltpu
from jax.experimental.pallas import tpu_sc as plsc

N_TAG = 1000
N_VID = 10000
D = 512
E = 75000

NC, NS, L = 2, 16, 16
EPAD = 75008
EPT = EPAD // NS
VREGS = EPT // L
IROWS = 37
IDX_ROWS = 2 * IROWS
NV_PAD = 10240
SLAB = 983040
CHUNKS = 11
PAD_PER_TILE = 1024
SLAB_TOTAL = SLAB + NS * PAD_PER_TILE
FLAT = N_TAG * NV_PAD
WPT = SLAB // NS
ZB = 4096
assert WPT % ZB == 0


def _sc_body(ps, pd, ng, nd, labels_out, cntall_out,
             slab, fpos, fneg, temp, idx2, ones_v, zeros_v,
             bounce_a, bounce_b, sem_a, sem_scat, sem_out, sem_z):
    bounce = (bounce_a, bounce_b)
    cid = lax.axis_index("c")
    sid = lax.axis_index("s")
    ebase = sid * EPT
    iota = lax.iota(jnp.int32, L)
    dumbase = SLAB + sid * PAD_PER_TILE

    ones16 = jnp.ones((L,), jnp.float32)
    zero16 = jnp.zeros((L,), jnp.float32)
    for t in range(128 // L):
        ones_v[pl.ds(t * L, L)] = ones16

    def zfill(j, c):
        zeros_v[pl.ds(j * L, L)] = zero16
        return c
    lax.fori_loop(0, ZB // L, zfill, 0)

    def load_flat(src_hbm, dst_hbm, out_ref):
        h1 = pltpu.async_copy(src_hbm.at[pl.ds(ebase, EPT)], out_ref, sem_a)
        h2 = pltpu.async_copy(dst_hbm.at[pl.ds(ebase, EPT)], temp, sem_a)
        h1.wait()
        h2.wait()

        def flat(j, c):
            t = out_ref[pl.ds(j * L, L)]
            v = temp[pl.ds(j * L, L)]
            out_ref[pl.ds(j * L, L)] = t * NV_PAD + v
            return c
        lax.fori_loop(0, VREGS, flat, 0)

    load_flat(ps, pd, fpos)
    load_flat(ng, nd, fneg)

    def build_idx(buf, row0, lo):
        hi = lo + SLAB

        def bd(j, c):
            f = buf[pl.ds(j * L, L)]
            ok = (f >= lo) & (f < hi)
            dum = dumbase + ((j * L) & (PAD_PER_TILE - 1)) + iota
            idx2[row0 + (j >> 3), pl.ds((j & 7) * L, L)] = jnp.where(ok, f - lo, dum)
            return c
        lax.fori_loop(0, VREGS, bd, 0)
        for t in range(VREGS, IROWS * 8):
            idx2[row0 + t // 8, pl.ds((t % 8) * L, L)] = (
                dumbase + ((t * L) & (PAD_PER_TILE - 1)) + iota)

    def run_phase(target, include_neg, parity):
        def task(t, c):
            chunk = 2 * t + parity
            lo = jnp.minimum(chunk * SLAB, FLAT - SLAB)

            build_idx(fpos, 0, lo)
            if include_neg:
                build_idx(fneg, IROWS, lo)
            nrows = IDX_ROWS if include_neg else IROWS

            def fire(r, c2):
                pltpu.async_copy(ones_v, slab.at[idx2.at[r]], sem_scat, add=True)
                return c2
            lax.fori_loop(0, nrows, fire, 0)

            def drain(r, c2):
                pltpu.make_async_copy(ones_v, slab.at[idx2.at[0]], sem_scat).wait()
                return c2
            lax.fori_loop(0, nrows, drain, 0)
            plsc.subcore_barrier()

            houts = [None, None]
            zs = []
            for k in range(WPT // ZB):
                b = k % 2
                if houts[b] is not None:
                    houts[b].wait()
                pltpu.async_copy(slab.at[pl.ds(sid * WPT + k * ZB, ZB)],
                                 bounce[b], sem_a).wait()
                zs.append(pltpu.async_copy(
                    zeros_v, slab.at[pl.ds(sid * WPT + k * ZB, ZB)], sem_z))
                houts[b] = pltpu.async_copy(
                    bounce[b],
                    target.at[pl.ds(lo + sid * WPT + k * ZB, ZB)],
                    sem_out)
            for h in houts:
                h.wait()
            for h in zs:
                h.wait()
            plsc.subcore_barrier()
            return c

        lax.fori_loop(0, (CHUNKS + 1 - parity) // 2, task, 0)

    zh = [pltpu.async_copy(zeros_v, slab.at[pl.ds(sid * WPT + k * ZB, ZB)], sem_a)
          for k in range(WPT // ZB)]
    for h in zh:
        h.wait()
    plsc.subcore_barrier()

    run_phase(cntall_out, True, cid)
    run_phase(labels_out, False, 1 - cid)


_sc_counts = pl.kernel(
    _sc_body,
    out_type=[jax.ShapeDtypeStruct((FLAT,), jnp.float32),
              jax.ShapeDtypeStruct((FLAT,), jnp.float32)],
    mesh=plsc.VectorSubcoreMesh(core_axis_name="c", subcore_axis_name="s",
                                num_cores=NC, num_subcores=NS),
    scratch_types=[
        pltpu.VMEM_SHARED((SLAB_TOTAL,), jnp.float32),
        pltpu.VMEM((EPT,), jnp.int32),
        pltpu.VMEM((EPT,), jnp.int32),
        pltpu.VMEM((EPT,), jnp.int32),
        pltpu.VMEM((IDX_ROWS + 6, 128), jnp.int32),
        pltpu.VMEM((128,), jnp.float32),
        pltpu.VMEM((ZB,), jnp.float32),
        pltpu.VMEM((ZB,), jnp.float32),
        pltpu.VMEM((ZB,), jnp.float32),
        pltpu.SemaphoreType.DMA,
        pltpu.SemaphoreType.DMA,
        pltpu.SemaphoreType.DMA,
        pltpu.SemaphoreType.DMA,
    ],
)


BM = 1024
BV = BM // 128


def _tc_body(hv_ref, ht_ref, cnt_ref, lab_ref, out_ref, lab2_ref):
    acc = lax.dot_general(ht_ref[...], hv_ref[...], (((1,), (1,)), ((), ())),
                          preferred_element_type=jnp.float32,
                          precision=lax.Precision.DEFAULT)
    out_ref[...] = acc * cnt_ref[...].reshape(N_TAG, BM)
    lab2_ref[...] = lab_ref[...].reshape(N_TAG, BM)


def _tc_score(h_video, h_tag, cnt_flat, lab_flat):
    cnt3 = cnt_flat.reshape(N_TAG, NV_PAD // 128, 128)
    lab3 = lab_flat.reshape(N_TAG, NV_PAD // 128, 128)
    return pl.pallas_call(
        _tc_body,
        grid=(NV_PAD // BM,),
        in_specs=[pl.BlockSpec((BM, D), lambda i: (i, 0)),
                  pl.BlockSpec((N_TAG, D), lambda i: (0, 0)),
                  pl.BlockSpec((N_TAG, BV, 128), lambda i: (0, i, 0)),
                  pl.BlockSpec((N_TAG, BV, 128), lambda i: (0, i, 0))],
        out_specs=[pl.BlockSpec((N_TAG, BM), lambda i: (0, i)),
                   pl.BlockSpec((N_TAG, BM), lambda i: (0, i))],
        out_shape=[jax.ShapeDtypeStruct((N_TAG, N_VID), jnp.float32),
                   jax.ShapeDtypeStruct((N_TAG, N_VID), jnp.float32)],
    )(h_video, h_tag, cnt3, lab3)


def kernel(h_tag, h_video, pos_src, pos_dst, neg_src, neg_dst):
    npad = EPAD - E
    pz = jnp.zeros((npad,), jnp.int32)
    pv = jnp.full((npad,), N_VID, jnp.int32)
    ps = jnp.concatenate([pos_src.astype(jnp.int32), pz])
    pd = jnp.concatenate([pos_dst.astype(jnp.int32), pv])
    ng = jnp.concatenate([neg_src.astype(jnp.int32), pz])
    nd = jnp.concatenate([neg_dst.astype(jnp.int32), pv])

    labels_flat, cntall_flat = _sc_counts(ps, pd, ng, nd)
    cls_t, labels_t = _tc_score(h_video, h_tag, cntall_flat, labels_flat)
    return cls_t.T, labels_t.T


# speedup vs baseline: 9.7132x; 1.0782x over previous
import jax
import jax.numpy as jnp
from jax import lax
from jax.experimental import pallas as pl
from jax.experimental.pallas import tpu as pltpu
from jax.experimental.pallas import tpu_sc as plsc

N_TAG = 1000
N_VID = 10000
D = 512
E = 75000

NC, NS, L = 2, 16, 16
EPAD = 75008
EPT = EPAD // NS
VREGS = EPT // L
IROWS = 37
IDX_ROWS = 2 * IROWS
NV_PAD = 10240
SLAB = 1 << 20
CHUNKS = 10
PAD_PER_TILE = 1024
SLAB_TOTAL = SLAB + NS * PAD_PER_TILE
FLAT = N_TAG * NV_PAD
WPT = SLAB // NS
ZB = 4096
assert WPT % ZB == 0


def _sc_body(ps, pd, ng, nd, labels_out, cntall_out,
             slab, fpos, fneg, temp, idx2, ones_v, zeros_v,
             bounce_a, bounce_b, sem_a, sem_scat, sem_out, sem_z):
    bounce = (bounce_a, bounce_b)
    cid = lax.axis_index("c")
    sid = lax.axis_index("s")
    ebase = sid * EPT
    iota = lax.iota(jnp.int32, L)
    dumbase = SLAB + sid * PAD_PER_TILE

    ones16 = jnp.ones((L,), jnp.float32)
    zero16 = jnp.zeros((L,), jnp.float32)
    for t in range(128 // L):
        ones_v[pl.ds(t * L, L)] = ones16

    def zfill(j, c):
        zeros_v[pl.ds(j * L, L)] = zero16
        return c
    lax.fori_loop(0, ZB // L, zfill, 0)

    def load_flat(src_hbm, dst_hbm, out_ref):
        h1 = pltpu.async_copy(src_hbm.at[pl.ds(ebase, EPT)], out_ref, sem_a)
        h2 = pltpu.async_copy(dst_hbm.at[pl.ds(ebase, EPT)], temp, sem_a)
        h1.wait()
        h2.wait()

        def flat(j, c):
            t = out_ref[pl.ds(j * L, L)]
            v = temp[pl.ds(j * L, L)]
            out_ref[pl.ds(j * L, L)] = t * NV_PAD + v
            return c
        lax.fori_loop(0, VREGS, flat, 0)

    load_flat(ps, pd, fpos)
    load_flat(ng, nd, fneg)

    def build_idx(buf, row0, lo):
        hi = lo + SLAB

        def bd(j, c):
            f = buf[pl.ds(j * L, L)]
            ok = (f >= lo) & (f < hi)
            dum = dumbase + ((j * L) & (PAD_PER_TILE - 1)) + iota
            idx2[row0 + (j >> 3), pl.ds((j & 7) * L, L)] = jnp.where(ok, f - lo, dum)
            return c
        lax.fori_loop(0, VREGS, bd, 0)
        for t in range(VREGS, IROWS * 8):
            idx2[row0 + t // 8, pl.ds((t % 8) * L, L)] = (
                dumbase + ((t * L) & (PAD_PER_TILE - 1)) + iota)

    def scatter_rows(nrows):
        def fire(r, c2):
            pltpu.async_copy(ones_v, slab.at[idx2.at[r]], sem_scat, add=True)
            return c2
        lax.fori_loop(0, nrows, fire, 0)

        def drain(r, c2):
            pltpu.make_async_copy(ones_v, slab.at[idx2.at[0]], sem_scat).wait()
            return c2
        lax.fori_loop(0, nrows, drain, 0)
        plsc.subcore_barrier()

    def copyout_rezero(target, lo):
        houts = [None, None]
        zs = []
        for k in range(WPT // ZB):
            b = k % 2
            if houts[b] is not None:
                houts[b].wait()
            pltpu.async_copy(slab.at[pl.ds(sid * WPT + k * ZB, ZB)],
                             bounce[b], sem_a).wait()
            zs.append(pltpu.async_copy(
                zeros_v, slab.at[pl.ds(sid * WPT + k * ZB, ZB)], sem_z))
            houts[b] = pltpu.async_copy(
                bounce[b],
                target.at[pl.ds(lo + sid * WPT + k * ZB, ZB)],
                sem_out)
        for h in houts:
            h.wait()
        for h in zs:
            h.wait()
        plsc.subcore_barrier()

    zh = [pltpu.async_copy(zeros_v, slab.at[pl.ds(sid * WPT + k * ZB, ZB)], sem_a)
          for k in range(WPT // ZB)]
    for h in zh:
        h.wait()
    plsc.subcore_barrier()

    def task(t, c):
        chunk = 2 * t + cid
        lo = jnp.minimum(chunk * SLAB, FLAT - SLAB)
        build_idx(fpos, 0, lo)
        build_idx(fneg, IROWS, lo)
        scatter_rows(IDX_ROWS)
        copyout_rezero(cntall_out, lo)
        scatter_rows(IROWS)
        copyout_rezero(labels_out, lo)
        return c

    lax.fori_loop(0, (CHUNKS + 1 - cid) // 2, task, 0)


_sc_counts = pl.kernel(
    _sc_body,
    out_type=[jax.ShapeDtypeStruct((FLAT,), jnp.float32),
              jax.ShapeDtypeStruct((FLAT,), jnp.float32)],
    mesh=plsc.VectorSubcoreMesh(core_axis_name="c", subcore_axis_name="s",
                                num_cores=NC, num_subcores=NS),
    scratch_types=[
        pltpu.VMEM_SHARED((SLAB_TOTAL,), jnp.float32),
        pltpu.VMEM((EPT,), jnp.int32),
        pltpu.VMEM((EPT,), jnp.int32),
        pltpu.VMEM((EPT,), jnp.int32),
        pltpu.VMEM((IDX_ROWS + 6, 128), jnp.int32),
        pltpu.VMEM((128,), jnp.float32),
        pltpu.VMEM((ZB,), jnp.float32),
        pltpu.VMEM((ZB,), jnp.float32),
        pltpu.VMEM((ZB,), jnp.float32),
        pltpu.SemaphoreType.DMA,
        pltpu.SemaphoreType.DMA,
        pltpu.SemaphoreType.DMA,
        pltpu.SemaphoreType.DMA,
    ],
)


BM = 1024
BV = BM // 128


def _tc_body(hv_ref, ht_ref, cnt_ref, lab_ref, out_ref, lab2_ref):
    acc = lax.dot_general(ht_ref[...], hv_ref[...], (((1,), (1,)), ((), ())),
                          preferred_element_type=jnp.float32,
                          precision=lax.Precision.DEFAULT)
    out_ref[...] = acc * cnt_ref[...].reshape(N_TAG, BM)
    lab2_ref[...] = lab_ref[...].reshape(N_TAG, BM)


def _tc_score(h_video, h_tag, cnt_flat, lab_flat):
    cnt3 = cnt_flat.reshape(N_TAG, NV_PAD // 128, 128)
    lab3 = lab_flat.reshape(N_TAG, NV_PAD // 128, 128)
    return pl.pallas_call(
        _tc_body,
        grid=(NV_PAD // BM,),
        in_specs=[pl.BlockSpec((BM, D), lambda i: (i, 0)),
                  pl.BlockSpec((N_TAG, D), lambda i: (0, 0)),
                  pl.BlockSpec((N_TAG, BV, 128), lambda i: (0, i, 0)),
                  pl.BlockSpec((N_TAG, BV, 128), lambda i: (0, i, 0))],
        out_specs=[pl.BlockSpec((N_TAG, BM), lambda i: (0, i)),
                   pl.BlockSpec((N_TAG, BM), lambda i: (0, i))],
        out_shape=[jax.ShapeDtypeStruct((N_TAG, N_VID), jnp.float32),
                   jax.ShapeDtypeStruct((N_TAG, N_VID), jnp.float32)],
    )(h_video, h_tag, cnt3, lab3)


def kernel(h_tag, h_video, pos_src, pos_dst, neg_src, neg_dst):
    npad = EPAD - E
    pz = jnp.zeros((npad,), jnp.int32)
    pv = jnp.full((npad,), N_VID, jnp.int32)
    ps = jnp.concatenate([pos_src.astype(jnp.int32), pz])
    pd = jnp.concatenate([pos_dst.astype(jnp.int32), pv])
    ng = jnp.concatenate([neg_src.astype(jnp.int32), pz])
    nd = jnp.concatenate([neg_dst.astype(jnp.int32), pv])

    labels_flat, cntall_flat = _sc_counts(ps, pd, ng, nd)
    cls_t, labels_t = _tc_score(h_video, h_tag, cntall_flat, labels_flat)
    return cls_t.T, labels_t.T


# speedup vs baseline: 9.7225x; 1.0010x over previous
import jax
import jax.numpy as jnp
from jax import lax
from jax.experimental import pallas as pl
from jax.experimental.pallas import tpu as pltpu
from jax.experimental.pallas import tpu_sc as plsc

N_TAG = 1000
N_VID = 10000
D = 512
E = 75000

NC, NS, L = 2, 16, 16
EPAD = 75008
EPT = EPAD // NS
VREGS = EPT // L
IROWS = 37
IDX_ROWS = 2 * IROWS
NV_PAD = 10240
SLAB = 1 << 20
CHUNKS = 10
PAD_PER_TILE = 1024
SLAB_TOTAL = SLAB + NS * PAD_PER_TILE
FLAT = N_TAG * NV_PAD
WPT = SLAB // NS
ZB = 4096
assert WPT % ZB == 0


def _sc_body(ps, pd, ng, nd, labels_out, cntall_out,
             slab, fpos, fneg, temp, idx2, ones_v, zeros_v,
             bounce_a, bounce_b, sem_a, sem_scat, sem_out, sem_z):
    bounce = (bounce_a, bounce_b)
    cid = lax.axis_index("c")
    sid = lax.axis_index("s")
    ebase = sid * EPT
    iota = lax.iota(jnp.int32, L)
    dumbase = SLAB + sid * PAD_PER_TILE

    ones16 = jnp.ones((L,), jnp.float32)
    zero16 = jnp.zeros((L,), jnp.float32)
    for t in range(128 // L):
        ones_v[pl.ds(t * L, L)] = ones16

    def zfill(j, c):
        zeros_v[pl.ds(j * L, L)] = zero16
        return c
    lax.fori_loop(0, ZB // L, zfill, 0)

    def load_flat(src_hbm, dst_hbm, out_ref):
        h1 = pltpu.async_copy(src_hbm.at[pl.ds(ebase, EPT)], out_ref, sem_a)
        h2 = pltpu.async_copy(dst_hbm.at[pl.ds(ebase, EPT)], temp, sem_a)
        h1.wait()
        h2.wait()

        def flat(j, c):
            t = out_ref[pl.ds(j * L, L)]
            v = temp[pl.ds(j * L, L)]
            out_ref[pl.ds(j * L, L)] = t * NV_PAD + v
            return c
        lax.fori_loop(0, VREGS, flat, 0)

    load_flat(ps, pd, fpos)
    load_flat(ng, nd, fneg)

    def build_idx(buf, row0, lo):
        hi = lo + SLAB

        def bd(j, c):
            f = buf[pl.ds(j * L, L)]
            ok = (f >= lo) & (f < hi)
            dum = dumbase + ((j * L) & (PAD_PER_TILE - 1)) + iota
            idx2[row0 + (j >> 3), pl.ds((j & 7) * L, L)] = jnp.where(ok, f - lo, dum)
            return c
        lax.fori_loop(0, VREGS, bd, 0)
        for t in range(VREGS, IROWS * 8):
            idx2[row0 + t // 8, pl.ds((t % 8) * L, L)] = (
                dumbase + ((t * L) & (PAD_PER_TILE - 1)) + iota)

    def scatter_rows(nrows):
        def fire(r, c2):
            pltpu.async_copy(ones_v, slab.at[idx2.at[r]], sem_scat, add=True)
            return c2
        lax.fori_loop(0, nrows, fire, 0)

        def drain(r, c2):
            pltpu.make_async_copy(ones_v, slab.at[idx2.at[0]], sem_scat).wait()
            return c2
        lax.fori_loop(0, nrows, drain, 0)
        plsc.subcore_barrier()

    def copyout_rezero(target, lo):
        nseg = WPT // ZB
        houts = [None, None]
        rds = [None, None]
        zs = []
        rds[0] = pltpu.async_copy(slab.at[pl.ds(sid * WPT, ZB)],
                                  bounce[0], sem_a)
        for k in range(nseg):
            b = k % 2
            rds[b].wait()
            zs.append(pltpu.async_copy(
                zeros_v, slab.at[pl.ds(sid * WPT + k * ZB, ZB)], sem_z))
            if k + 1 < nseg:
                nb = (k + 1) % 2
                if houts[nb] is not None:
                    houts[nb].wait()
                rds[nb] = pltpu.async_copy(
                    slab.at[pl.ds(sid * WPT + (k + 1) * ZB, ZB)],
                    bounce[nb], sem_a)
            houts[b] = pltpu.async_copy(
                bounce[b],
                target.at[pl.ds(lo + sid * WPT + k * ZB, ZB)],
                sem_out)
        for h in houts:
            h.wait()
        for h in zs:
            h.wait()
        plsc.subcore_barrier()

    zh = [pltpu.async_copy(zeros_v, slab.at[pl.ds(sid * WPT + k * ZB, ZB)], sem_a)
          for k in range(WPT // ZB)]
    for h in zh:
        h.wait()
    plsc.subcore_barrier()

    def task(t, c):
        chunk = 2 * t + cid
        lo = jnp.minimum(chunk * SLAB, FLAT - SLAB)
        build_idx(fpos, 0, lo)
        build_idx(fneg, IROWS, lo)
        scatter_rows(IDX_ROWS)
        copyout_rezero(cntall_out, lo)
        scatter_rows(IROWS)
        copyout_rezero(labels_out, lo)
        return c

    lax.fori_loop(0, (CHUNKS + 1 - cid) // 2, task, 0)


_sc_counts = pl.kernel(
    _sc_body,
    out_type=[jax.ShapeDtypeStruct((FLAT,), jnp.float32),
              jax.ShapeDtypeStruct((FLAT,), jnp.float32)],
    mesh=plsc.VectorSubcoreMesh(core_axis_name="c", subcore_axis_name="s",
                                num_cores=NC, num_subcores=NS),
    scratch_types=[
        pltpu.VMEM_SHARED((SLAB_TOTAL,), jnp.float32),
        pltpu.VMEM((EPT,), jnp.int32),
        pltpu.VMEM((EPT,), jnp.int32),
        pltpu.VMEM((EPT,), jnp.int32),
        pltpu.VMEM((IDX_ROWS + 6, 128), jnp.int32),
        pltpu.VMEM((128,), jnp.float32),
        pltpu.VMEM((ZB,), jnp.float32),
        pltpu.VMEM((ZB,), jnp.float32),
        pltpu.VMEM((ZB,), jnp.float32),
        pltpu.SemaphoreType.DMA,
        pltpu.SemaphoreType.DMA,
        pltpu.SemaphoreType.DMA,
        pltpu.SemaphoreType.DMA,
    ],
)


BM = 1024
BV = BM // 128


def _tc_body(hv_ref, ht_ref, cnt_ref, lab_ref, out_ref, lab2_ref):
    acc = lax.dot_general(ht_ref[...], hv_ref[...], (((1,), (1,)), ((), ())),
                          preferred_element_type=jnp.float32,
                          precision=lax.Precision.DEFAULT)
    out_ref[...] = acc * cnt_ref[...].reshape(N_TAG, BM)
    lab2_ref[...] = lab_ref[...].reshape(N_TAG, BM)


def _tc_score(h_video, h_tag, cnt_flat, lab_flat):
    cnt3 = cnt_flat.reshape(N_TAG, NV_PAD // 128, 128)
    lab3 = lab_flat.reshape(N_TAG, NV_PAD // 128, 128)
    return pl.pallas_call(
        _tc_body,
        grid=(NV_PAD // BM,),
        in_specs=[pl.BlockSpec((BM, D), lambda i: (i, 0)),
                  pl.BlockSpec((N_TAG, D), lambda i: (0, 0)),
                  pl.BlockSpec((N_TAG, BV, 128), lambda i: (0, i, 0)),
                  pl.BlockSpec((N_TAG, BV, 128), lambda i: (0, i, 0))],
        out_specs=[pl.BlockSpec((N_TAG, BM), lambda i: (0, i)),
                   pl.BlockSpec((N_TAG, BM), lambda i: (0, i))],
        out_shape=[jax.ShapeDtypeStruct((N_TAG, N_VID), jnp.float32),
                   jax.ShapeDtypeStruct((N_TAG, N_VID), jnp.float32)],
    )(h_video, h_tag, cnt3, lab3)


def kernel(h_tag, h_video, pos_src, pos_dst, neg_src, neg_dst):
    npad = EPAD - E
    pz = jnp.zeros((npad,), jnp.int32)
    pv = jnp.full((npad,), N_VID, jnp.int32)
    ps = jnp.concatenate([pos_src.astype(jnp.int32), pz])
    pd = jnp.concatenate([pos_dst.astype(jnp.int32), pv])
    ng = jnp.concatenate([neg_src.astype(jnp.int32), pz])
    nd = jnp.concatenate([neg_dst.astype(jnp.int32), pv])

    labels_flat, cntall_flat = _sc_counts(ps, pd, ng, nd)
    cls_t, labels_t = _tc_score(h_video, h_tag, cntall_flat, labels_flat)
    return cls_t.T, labels_t.T


# speedup vs baseline: 9.8940x; 1.0176x over previous
import jax
import jax.numpy as jnp
from jax import lax
from jax.experimental import pallas as pl
from jax.experimental.pallas import tpu as pltpu
from jax.experimental.pallas import tpu_sc as plsc

N_TAG = 1000
N_VID = 10000
D = 512
E = 75000

NC, NS, L = 2, 16, 16
EPAD = 75008
EPT = EPAD // NS
VREGS = EPT // L
IROWS = 37
IDX_ROWS = 2 * IROWS
NV_PAD = 10240
SLAB = 1 << 20
CHUNKS = 10
PAD_PER_TILE = 1024
SLAB_TOTAL = SLAB + NS * PAD_PER_TILE
FLAT = N_TAG * NV_PAD
WPT = SLAB // NS
ZB = 4096
assert WPT % ZB == 0


def _sc_body(ps, pd, ng, nd, labels_out, cntall_out,
             slab, fpos, fneg, temp, idx2, ones_v, zeros_v,
             bounce_a, bounce_b, sem_a, sem_scat, sem_out, sem_z):
    bounce = (bounce_a, bounce_b)
    cid = lax.axis_index("c")
    sid = lax.axis_index("s")
    ebase = sid * EPT
    iota = lax.iota(jnp.int32, L)
    dumbase = SLAB + sid * PAD_PER_TILE

    ones16 = jnp.ones((L,), jnp.float32)
    zero16 = jnp.zeros((L,), jnp.float32)
    for t in range(128 // L):
        ones_v[pl.ds(t * L, L)] = ones16

    def zfill(j, c):
        zeros_v[pl.ds(j * L, L)] = zero16
        return c
    lax.fori_loop(0, ZB // L, zfill, 0)

    def load_flat(src_hbm, dst_hbm, out_ref):
        h1 = pltpu.async_copy(src_hbm.at[pl.ds(ebase, EPT)], out_ref, sem_a)
        h2 = pltpu.async_copy(dst_hbm.at[pl.ds(ebase, EPT)], temp, sem_a)
        h1.wait()
        h2.wait()

        def flat(j, c):
            t = out_ref[pl.ds(j * L, L)]
            v = temp[pl.ds(j * L, L)]
            valid = (ebase + j * L) + iota < E
            out_ref[pl.ds(j * L, L)] = jnp.where(valid, t * NV_PAD + v, FLAT)
            return c
        lax.fori_loop(0, VREGS, flat, 0)

    load_flat(ps, pd, fpos)
    load_flat(ng, nd, fneg)

    def build_idx(buf, row0, lo):
        hi = lo + SLAB

        def bd(j, c):
            f = buf[pl.ds(j * L, L)]
            ok = (f >= lo) & (f < hi)
            dum = dumbase + ((j * L) & (PAD_PER_TILE - 1)) + iota
            idx2[row0 + (j >> 3), pl.ds((j & 7) * L, L)] = jnp.where(ok, f - lo, dum)
            return c
        lax.fori_loop(0, VREGS, bd, 0)
        for t in range(VREGS, IROWS * 8):
            idx2[row0 + t // 8, pl.ds((t % 8) * L, L)] = (
                dumbase + ((t * L) & (PAD_PER_TILE - 1)) + iota)

    def scatter_rows(nrows):
        def fire(r, c2):
            pltpu.async_copy(ones_v, slab.at[idx2.at[r]], sem_scat, add=True)
            return c2
        lax.fori_loop(0, nrows, fire, 0)

        def drain(r, c2):
            pltpu.make_async_copy(ones_v, slab.at[idx2.at[0]], sem_scat).wait()
            return c2
        lax.fori_loop(0, nrows, drain, 0)
        plsc.subcore_barrier()

    def copyout_rezero(target, lo):
        nseg = WPT // ZB
        houts = [None, None]
        rds = [None, None]
        zs = []
        rds[0] = pltpu.async_copy(slab.at[pl.ds(sid * WPT, ZB)],
                                  bounce[0], sem_a)
        for k in range(nseg):
            b = k % 2
            rds[b].wait()
            zs.append(pltpu.async_copy(
                zeros_v, slab.at[pl.ds(sid * WPT + k * ZB, ZB)], sem_z))
            if k + 1 < nseg:
                nb = (k + 1) % 2
                if houts[nb] is not None:
                    houts[nb].wait()
                rds[nb] = pltpu.async_copy(
                    slab.at[pl.ds(sid * WPT + (k + 1) * ZB, ZB)],
                    bounce[nb], sem_a)
            houts[b] = pltpu.async_copy(
                bounce[b],
                target.at[pl.ds(lo + sid * WPT + k * ZB, ZB)],
                sem_out)
        for h in houts:
            h.wait()
        for h in zs:
            h.wait()
        plsc.subcore_barrier()

    zh = [pltpu.async_copy(zeros_v, slab.at[pl.ds(sid * WPT + k * ZB, ZB)], sem_a)
          for k in range(WPT // ZB)]
    for h in zh:
        h.wait()
    plsc.subcore_barrier()

    def task(t, c):
        chunk = 2 * t + cid
        lo = jnp.minimum(chunk * SLAB, FLAT - SLAB)
        build_idx(fpos, 0, lo)
        build_idx(fneg, IROWS, lo)
        scatter_rows(IDX_ROWS)
        copyout_rezero(cntall_out, lo)
        scatter_rows(IROWS)
        copyout_rezero(labels_out, lo)
        return c

    lax.fori_loop(0, (CHUNKS + 1 - cid) // 2, task, 0)


_sc_counts = pl.kernel(
    _sc_body,
    out_type=[jax.ShapeDtypeStruct((FLAT,), jnp.float32),
              jax.ShapeDtypeStruct((FLAT,), jnp.float32)],
    mesh=plsc.VectorSubcoreMesh(core_axis_name="c", subcore_axis_name="s",
                                num_cores=NC, num_subcores=NS),
    scratch_types=[
        pltpu.VMEM_SHARED((SLAB_TOTAL,), jnp.float32),
        pltpu.VMEM((EPT,), jnp.int32),
        pltpu.VMEM((EPT,), jnp.int32),
        pltpu.VMEM((EPT,), jnp.int32),
        pltpu.VMEM((IDX_ROWS + 6, 128), jnp.int32),
        pltpu.VMEM((128,), jnp.float32),
        pltpu.VMEM((ZB,), jnp.float32),
        pltpu.VMEM((ZB,), jnp.float32),
        pltpu.VMEM((ZB,), jnp.float32),
        pltpu.SemaphoreType.DMA,
        pltpu.SemaphoreType.DMA,
        pltpu.SemaphoreType.DMA,
        pltpu.SemaphoreType.DMA,
    ],
)


BM = 1024
BV = BM // 128


def _tc_body(hv_ref, ht_ref, cnt_ref, lab_ref, out_ref, lab2_ref):
    acc = lax.dot_general(ht_ref[...], hv_ref[...], (((1,), (1,)), ((), ())),
                          preferred_element_type=jnp.float32,
                          precision=lax.Precision.DEFAULT)
    out_ref[...] = acc * cnt_ref[...].reshape(N_TAG, BM)
    lab2_ref[...] = lab_ref[...].reshape(N_TAG, BM)


def _tc_score(h_video, h_tag, cnt_flat, lab_flat):
    cnt3 = cnt_flat.reshape(N_TAG, NV_PAD // 128, 128)
    lab3 = lab_flat.reshape(N_TAG, NV_PAD // 128, 128)
    return pl.pallas_call(
        _tc_body,
        grid=(NV_PAD // BM,),
        in_specs=[pl.BlockSpec((BM, D), lambda i: (i, 0)),
                  pl.BlockSpec((N_TAG, D), lambda i: (0, 0)),
                  pl.BlockSpec((N_TAG, BV, 128), lambda i: (0, i, 0)),
                  pl.BlockSpec((N_TAG, BV, 128), lambda i: (0, i, 0))],
        out_specs=[pl.BlockSpec((N_TAG, BM), lambda i: (0, i)),
                   pl.BlockSpec((N_TAG, BM), lambda i: (0, i))],
        out_shape=[jax.ShapeDtypeStruct((N_TAG, N_VID), jnp.float32),
                   jax.ShapeDtypeStruct((N_TAG, N_VID), jnp.float32)],
    )(h_video, h_tag, cnt3, lab3)


def kernel(h_tag, h_video, pos_src, pos_dst, neg_src, neg_dst):
    labels_flat, cntall_flat = _sc_counts(
        pos_src.astype(jnp.int32), pos_dst.astype(jnp.int32),
        neg_src.astype(jnp.int32), neg_dst.astype(jnp.int32))
    cls_t, labels_t = _tc_score(h_video, h_tag, cntall_flat, labels_flat)
    return cls_t.T, labels_t.T


# speedup vs baseline: 10.8668x; 1.0983x over previous
import jax
import jax.numpy as jnp
from jax import lax
from jax.experimental import pallas as pl
from jax.experimental.pallas import tpu as pltpu
from jax.experimental.pallas import tpu_sc as plsc

N_TAG = 1000
N_VID = 10000
D = 512
E = 75000

NC, NS, L = 2, 16, 16
EPAD = 75008
EPT = EPAD // NS
VREGS = EPT // L
IROWS = 37
IDX_ROWS = 2 * IROWS
NV_PAD = 10240
SLAB = 1 << 20
CHUNKS = 10
PAD_PER_TILE = 1024
SLAB_TOTAL = SLAB + NS * PAD_PER_TILE
FLAT = N_TAG * NV_PAD
WPT = SLAB // NS
ZB = 4096
assert WPT % ZB == 0


def _sc_body(ps, pd, ng, nd, labels_out, cntall_out,
             slab, fpos, fneg, temp, idx2, ones_v, zeros_v,
             bounce_a, bounce_b, sem_a, sem_scat, sem_out, sem_z):
    bounce = (bounce_a, bounce_b)
    cid = lax.axis_index("c")
    sid = lax.axis_index("s")
    ebase = sid * EPT
    iota = lax.iota(jnp.int32, L)
    dumbase = SLAB + sid * PAD_PER_TILE

    ones16 = jnp.ones((L,), jnp.float32)
    zero16 = jnp.zeros((L,), jnp.float32)
    for t in range(128 // L):
        ones_v[pl.ds(t * L, L)] = ones16

    def zfill(j, c):
        zeros_v[pl.ds(j * L, L)] = zero16
        return c
    lax.fori_loop(0, ZB // L, zfill, 0)

    def load_flat(src_hbm, dst_hbm, out_ref):
        h1 = pltpu.async_copy(src_hbm.at[pl.ds(ebase, EPT)], out_ref, sem_a)
        h2 = pltpu.async_copy(dst_hbm.at[pl.ds(ebase, EPT)], temp, sem_a)
        h1.wait()
        h2.wait()

        def flat(j, c):
            t = out_ref[pl.ds(j * L, L)]
            v = temp[pl.ds(j * L, L)]
            valid = (ebase + j * L) + iota < E
            out_ref[pl.ds(j * L, L)] = jnp.where(valid, t * NV_PAD + v, FLAT)
            return c
        lax.fori_loop(0, VREGS, flat, 0)

    load_flat(ps, pd, fpos)
    load_flat(ng, nd, fneg)

    def build_idx(buf, row0, lo):
        hi = lo + SLAB

        def bd(j, c):
            f = buf[pl.ds(j * L, L)]
            ok = (f >= lo) & (f < hi)
            dum = dumbase + ((j * L) & (PAD_PER_TILE - 1)) + iota
            idx2[row0 + (j >> 3), pl.ds((j & 7) * L, L)] = jnp.where(ok, f - lo, dum)
            return c
        lax.fori_loop(0, VREGS, bd, 0)
        for t in range(VREGS, IROWS * 8):
            idx2[row0 + t // 8, pl.ds((t % 8) * L, L)] = (
                dumbase + ((t * L) & (PAD_PER_TILE - 1)) + iota)

    def scatter_rows(r0, r1):
        def fire(r, c2):
            pltpu.async_copy(ones_v, slab.at[idx2.at[r]], sem_scat, add=True)
            return c2
        lax.fori_loop(r0, r1, fire, 0)

        def drain(r, c2):
            pltpu.make_async_copy(ones_v, slab.at[idx2.at[0]], sem_scat).wait()
            return c2
        lax.fori_loop(r0, r1, drain, 0)
        plsc.subcore_barrier()

    def copyout(target, lo, rezero):
        nseg = WPT // ZB
        houts = [None, None]
        rds = [None, None]
        zs = []
        rds[0] = pltpu.async_copy(slab.at[pl.ds(sid * WPT, ZB)],
                                  bounce[0], sem_a)
        for k in range(nseg):
            b = k % 2
            rds[b].wait()
            if rezero:
                zs.append(pltpu.async_copy(
                    zeros_v, slab.at[pl.ds(sid * WPT + k * ZB, ZB)], sem_z))
            if k + 1 < nseg:
                nb = (k + 1) % 2
                if houts[nb] is not None:
                    houts[nb].wait()
                rds[nb] = pltpu.async_copy(
                    slab.at[pl.ds(sid * WPT + (k + 1) * ZB, ZB)],
                    bounce[nb], sem_a)
            houts[b] = pltpu.async_copy(
                bounce[b],
                target.at[pl.ds(lo + sid * WPT + k * ZB, ZB)],
                sem_out)
        for h in houts:
            h.wait()
        for h in zs:
            h.wait()
        plsc.subcore_barrier()

    zh = [pltpu.async_copy(zeros_v, slab.at[pl.ds(sid * WPT + k * ZB, ZB)], sem_a)
          for k in range(WPT // ZB)]
    for h in zh:
        h.wait()
    plsc.subcore_barrier()

    def task(t, c):
        chunk = 2 * t + cid
        lo = jnp.minimum(chunk * SLAB, FLAT - SLAB)
        build_idx(fpos, 0, lo)
        build_idx(fneg, IROWS, lo)
        scatter_rows(0, IROWS)
        copyout(labels_out, lo, rezero=False)
        scatter_rows(IROWS, IDX_ROWS)
        copyout(cntall_out, lo, rezero=True)
        return c

    lax.fori_loop(0, (CHUNKS + 1 - cid) // 2, task, 0)


_sc_counts = pl.kernel(
    _sc_body,
    out_type=[jax.ShapeDtypeStruct((FLAT,), jnp.float32),
              jax.ShapeDtypeStruct((FLAT,), jnp.float32)],
    mesh=plsc.VectorSubcoreMesh(core_axis_name="c", subcore_axis_name="s",
                                num_cores=NC, num_subcores=NS),
    scratch_types=[
        pltpu.VMEM_SHARED((SLAB_TOTAL,), jnp.float32),
        pltpu.VMEM((EPT,), jnp.int32),
        pltpu.VMEM((EPT,), jnp.int32),
        pltpu.VMEM((EPT,), jnp.int32),
        pltpu.VMEM((IDX_ROWS + 6, 128), jnp.int32),
        pltpu.VMEM((128,), jnp.float32),
        pltpu.VMEM((ZB,), jnp.float32),
        pltpu.VMEM((ZB,), jnp.float32),
        pltpu.VMEM((ZB,), jnp.float32),
        pltpu.SemaphoreType.DMA,
        pltpu.SemaphoreType.DMA,
        pltpu.SemaphoreType.DMA,
        pltpu.SemaphoreType.DMA,
    ],
)


BM = 1024
BV = BM // 128


def _tc_body(hv_ref, ht_ref, cnt_ref, lab_ref, out_ref, lab2_ref):
    acc = lax.dot_general(ht_ref[...], hv_ref[...], (((1,), (1,)), ((), ())),
                          preferred_element_type=jnp.float32,
                          precision=lax.Precision.DEFAULT)
    out_ref[...] = acc * cnt_ref[...].reshape(N_TAG, BM)
    lab2_ref[...] = lab_ref[...].reshape(N_TAG, BM)


def _tc_score(h_video, h_tag, cnt_flat, lab_flat):
    cnt3 = cnt_flat.reshape(N_TAG, NV_PAD // 128, 128)
    lab3 = lab_flat.reshape(N_TAG, NV_PAD // 128, 128)
    return pl.pallas_call(
        _tc_body,
        grid=(NV_PAD // BM,),
        in_specs=[pl.BlockSpec((BM, D), lambda i: (i, 0)),
                  pl.BlockSpec((N_TAG, D), lambda i: (0, 0)),
                  pl.BlockSpec((N_TAG, BV, 128), lambda i: (0, i, 0)),
                  pl.BlockSpec((N_TAG, BV, 128), lambda i: (0, i, 0))],
        out_specs=[pl.BlockSpec((N_TAG, BM), lambda i: (0, i)),
                   pl.BlockSpec((N_TAG, BM), lambda i: (0, i))],
        out_shape=[jax.ShapeDtypeStruct((N_TAG, N_VID), jnp.float32),
                   jax.ShapeDtypeStruct((N_TAG, N_VID), jnp.float32)],
    )(h_video, h_tag, cnt3, lab3)


def kernel(h_tag, h_video, pos_src, pos_dst, neg_src, neg_dst):
    labels_flat, cntall_flat = _sc_counts(
        pos_src.astype(jnp.int32), pos_dst.astype(jnp.int32),
        neg_src.astype(jnp.int32), neg_dst.astype(jnp.int32))
    cls_t, labels_t = _tc_score(h_video, h_tag, cntall_flat, labels_flat)
    return cls_t.T, labels_t.T
